# Initial kernel scaffold; baseline (speedup 1.0000x reference)
import jax
import jax.numpy as jnp
from jax.experimental import pallas as pl


def kernel(x, edge_index, batch, params):
    raise NotImplementedError("write your pallas kernel here")


# speedup vs baseline: 1.9465x; 1.9465x over previous
import functools

import jax
import jax.numpy as jnp
import numpy as np
from jax.experimental import pallas as pl
from jax.experimental.pallas import tpu as pltpu

_N = 10000
_G = 8
_BN = 1000


def _linear_body(x_ref, w_ref, b_ref, o_ref):
    o_ref[...] = (
        jnp.dot(x_ref[...], w_ref[...], preferred_element_type=jnp.float32)
        + b_ref[...]
    )


def _linear(x, Wc, bc):
    n, fin = x.shape
    fout = Wc.shape[1]
    grid = (n // _BN,)
    return pl.pallas_call(
        _linear_body,
        grid=grid,
        in_specs=[
            pl.BlockSpec((_BN, fin), lambda i: (i, 0)),
            pl.BlockSpec((fin, fout), lambda i: (0, 0)),
            pl.BlockSpec((1, fout), lambda i: (0, 0)),
        ],
        out_specs=pl.BlockSpec((_BN, fout), lambda i: (i, 0)),
        out_shape=jax.ShapeDtypeStruct((n, fout), jnp.float32),
    )(x, Wc, bc)


def _pool_body(h_ref, b1h_ref, gw_ref, gb_ref, fw_ref, fb_ref, o_ref):
    h = h_ref[...]
    gate = jnp.dot(h, gw_ref[...], preferred_element_type=jnp.float32)
    gate = gate + gb_ref[0, 0]
    seg = b1h_ref[...]
    gid = jax.lax.broadcasted_iota(jnp.int32, (_N, _G), 1)
    mask = gid == seg
    neg = jnp.float32(-1e30)
    m = jnp.max(jnp.where(mask, gate, neg), axis=0)
    m = jnp.where(m > jnp.float32(-1e29), m, 0.0)
    e = jnp.exp(gate - m[None, :]) * mask.astype(jnp.float32)
    s = jnp.sum(e, axis=0)
    w = e / (s[None, :] + 1e-16)
    pooled = jax.lax.dot_general(
        w, h, (((0,), (0,)), ((), ())),
        preferred_element_type=jnp.float32)
    o_ref[...] = (
        jnp.dot(pooled, fw_ref[...], preferred_element_type=jnp.float32)
        + fb_ref[...]
    )


def _pool_head(h, batch, gate_W, gate_b, fc_W, fc_b):
    b1h = batch.reshape(_N, 1)
    return pl.pallas_call(
        _pool_body,
        out_shape=jax.ShapeDtypeStruct((_G, 2), jnp.float32),
    )(h, b1h, gate_W.T, gate_b.reshape(1, 1), fc_W.T, fc_b.reshape(1, 2))


def _edge_phase(q, k, v, src, dst):
    d = q.shape[1]
    logits = jnp.sum(q[dst] * k[src], axis=-1) / np.sqrt(np.float32(d))
    m = jnp.max(logits)
    w = jnp.exp(logits - m)
    s = jax.ops.segment_sum(w, dst, num_segments=_N)
    agg = jax.ops.segment_sum(v[src] * w[:, None], dst, num_segments=_N)
    return agg, s


def kernel(x, edge_index, batch, params):
    src, dst = edge_index[0], edge_index[1]
    h = x
    for i in range(1, 5):
        p = params
        Wc = jnp.concatenate(
            [p["conv%d_W%s" % (i, n)].T for n in ("q", "k", "v", "s")], axis=1
        )
        bc = jnp.concatenate(
            [p["conv%d_b%s" % (i, n)] for n in ("q", "k", "v", "s")]
        ).reshape(1, -1)
        qkvs = _linear(h, Wc, bc)
        d = Wc.shape[1] // 4
        q, k_, v, skip = (qkvs[:, j * d:(j + 1) * d] for j in range(4))
        agg, s = _edge_phase(q, k_, v, src, dst)
        h = jax.nn.elu(agg / (s[:, None] + 1e-16) + skip)
    return _pool_head(
        h, batch, params["gate_W"], params["gate_b"],
        params["fc_W"], params["fc_b"],
    )


# speedup vs baseline: 2.7822x; 1.4294x over previous
import functools

import jax
import jax.numpy as jnp
import numpy as np
from jax import lax
from jax.experimental import pallas as pl
from jax.experimental.pallas import tpu as pltpu
from jax.experimental.pallas import tpu_sc as plsc

_N = 10000
_G = 8
_E = 320000
_BN = 1000

_NC = 2
_NS = 16
_NW = _NC * _NS
_EW = _E // _NW
_C = 80
_NCHUNK = _EW // _C


def _sc_mesh():
    return plsc.VectorSubcoreMesh(core_axis_name="c", subcore_axis_name="s")


@functools.lru_cache(maxsize=None)
def _make_edge_logits(d):
    @functools.partial(
        pl.kernel,
        mesh=_sc_mesh(),
        compiler_params=pltpu.CompilerParams(needs_layout_passes=False),
        out_type=jax.ShapeDtypeStruct((_E,), jnp.float32),
        scratch_types=[
            pltpu.VMEM((_C,), jnp.int32),
            pltpu.VMEM((_C,), jnp.int32),
            pltpu.VMEM((_C, d), jnp.float32),
            pltpu.VMEM((_C, d), jnp.float32),
            pltpu.VMEM((_C,), jnp.float32),
            pltpu.SemaphoreType.DMA,
        ],
    )
    def _k(q_hbm, k_hbm, src_hbm, dst_hbm, alpha_hbm,
           sidx, didx, qbuf, kbuf, abuf, sem):
        wid = lax.axis_index("s") * _NC + lax.axis_index("c")
        base = wid * _EW

        def chunk(ci, _):
            off = base + ci * _C
            pltpu.sync_copy(src_hbm.at[pl.ds(off, _C)], sidx)
            pltpu.sync_copy(dst_hbm.at[pl.ds(off, _C)], didx)
            pltpu.async_copy(q_hbm.at[didx], qbuf, sem).wait()
            pltpu.async_copy(k_hbm.at[sidx], kbuf, sem).wait()

            lanes = lax.broadcasted_iota(jnp.int32, (16,), 0)

            def egroup(eb, _):
                svec = jnp.zeros((16,), jnp.float32)
                for rr in range(16):
                    r = eb * 16 + rr
                    acc = qbuf[r, pl.ds(0, 16)] * kbuf[r, pl.ds(0, 16)]
                    for j in range(1, d // 16):
                        acc = acc + (qbuf[r, pl.ds(16 * j, 16)]
                                     * kbuf[r, pl.ds(16 * j, 16)])
                    tot = jnp.sum(acc, axis=0)
                    svec = jnp.where(lanes == rr, jnp.full((16,), tot), svec)
                abuf[pl.ds(eb * 16, 16)] = svec
                return 0

            lax.fori_loop(0, _C // 16, egroup, 0)
            pltpu.sync_copy(abuf, alpha_hbm.at[pl.ds(off, _C)])
            return 0

        lax.fori_loop(0, _NCHUNK, chunk, 0)

    return _k


def _edge_logits(q, k, src, dst):
    return _make_edge_logits(q.shape[1])(q, k, src, dst)


def _linear_body(x_ref, w_ref, b_ref, o_ref):
    o_ref[...] = (
        jnp.dot(x_ref[...], w_ref[...], preferred_element_type=jnp.float32)
        + b_ref[...]
    )


def _linear(x, Wc, bc):
    n, fin = x.shape
    fout = Wc.shape[1]
    grid = (n // _BN,)
    return pl.pallas_call(
        _linear_body,
        grid=grid,
        in_specs=[
            pl.BlockSpec((_BN, fin), lambda i: (i, 0)),
            pl.BlockSpec((fin, fout), lambda i: (0, 0)),
            pl.BlockSpec((1, fout), lambda i: (0, 0)),
        ],
        out_specs=pl.BlockSpec((_BN, fout), lambda i: (i, 0)),
        out_shape=jax.ShapeDtypeStruct((n, fout), jnp.float32),
    )(x, Wc, bc)


def _pool_body(h_ref, b1h_ref, gw_ref, gb_ref, fw_ref, fb_ref, o_ref):
    h = h_ref[...]
    gate = jnp.dot(h, gw_ref[...], preferred_element_type=jnp.float32)
    gate = gate + gb_ref[0, 0]
    seg = b1h_ref[...]
    gid = jax.lax.broadcasted_iota(jnp.int32, (_N, _G), 1)
    mask = gid == seg
    neg = jnp.float32(-1e30)
    m = jnp.max(jnp.where(mask, gate, neg), axis=0)
    m = jnp.where(m > jnp.float32(-1e29), m, 0.0)
    e = jnp.exp(gate - m[None, :]) * mask.astype(jnp.float32)
    s = jnp.sum(e, axis=0)
    w = e / (s[None, :] + 1e-16)
    pooled = jax.lax.dot_general(
        w, h, (((0,), (0,)), ((), ())),
        preferred_element_type=jnp.float32)
    o_ref[...] = (
        jnp.dot(pooled, fw_ref[...], preferred_element_type=jnp.float32)
        + fb_ref[...]
    )


def _pool_head(h, batch, gate_W, gate_b, fc_W, fc_b):
    b1h = batch.reshape(_N, 1)
    return pl.pallas_call(
        _pool_body,
        out_shape=jax.ShapeDtypeStruct((_G, 2), jnp.float32),
    )(h, b1h, gate_W.T, gate_b.reshape(1, 1), fc_W.T, fc_b.reshape(1, 2))


def _edge_phase(q, k, v, src, dst):
    d = q.shape[1]
    if d < 128:
        pad = ((0, 0), (0, 128 - d))
        qp, kp = jnp.pad(q, pad), jnp.pad(k, pad)
    else:
        qp, kp = q, k
    logits = _edge_logits(qp, kp, src, dst) / np.sqrt(np.float32(d))
    m = jnp.max(logits)
    w = jnp.exp(logits - m)
    s = jax.ops.segment_sum(w, dst, num_segments=_N)
    agg = jax.ops.segment_sum(v[src] * w[:, None], dst, num_segments=_N)
    return agg, s


def kernel(x, edge_index, batch, params):
    src, dst = edge_index[0], edge_index[1]
    h = x
    for i in range(1, 5):
        p = params
        Wc = jnp.concatenate(
            [p["conv%d_W%s" % (i, n)].T for n in ("q", "k", "v", "s")], axis=1
        )
        bc = jnp.concatenate(
            [p["conv%d_b%s" % (i, n)] for n in ("q", "k", "v", "s")]
        ).reshape(1, -1)
        qkvs = _linear(h, Wc, bc)
        d = Wc.shape[1] // 4
        q, k_, v, skip = (qkvs[:, j * d:(j + 1) * d] for j in range(4))
        agg, s = _edge_phase(q, k_, v, src, dst)
        h = jax.nn.elu(agg / (s[:, None] + 1e-16) + skip)
    return _pool_head(
        h, batch, params["gate_W"], params["gate_b"],
        params["fc_W"], params["fc_b"],
    )


# speedup vs baseline: 3.8383x; 1.3796x over previous
import functools

import jax
import jax.numpy as jnp
import numpy as np
from jax import lax
from jax.experimental import pallas as pl
from jax.experimental.pallas import tpu as pltpu
from jax.experimental.pallas import tpu_sc as plsc

_N = 10000
_G = 8
_E = 320000
_BN = 1000

_NC = 2
_NS = 16
_NW = _NC * _NS
_EW = _E // _NW
_C = 80
_NCHUNK = _EW // _C


def _sc_mesh():
    return plsc.VectorSubcoreMesh(core_axis_name="c", subcore_axis_name="s")


@functools.lru_cache(maxsize=None)
def _make_edge_logits(d):
    @functools.partial(
        pl.kernel,
        mesh=_sc_mesh(),
        compiler_params=pltpu.CompilerParams(needs_layout_passes=False),
        out_type=jax.ShapeDtypeStruct((_E,), jnp.float32),
        scratch_types=[
            pltpu.VMEM((_C,), jnp.int32),
            pltpu.VMEM((_C,), jnp.int32),
            pltpu.VMEM((_C, d), jnp.float32),
            pltpu.VMEM((_C, d), jnp.float32),
            pltpu.VMEM((_C,), jnp.float32),
            pltpu.SemaphoreType.DMA,
        ],
    )
    def _k(q_hbm, k_hbm, src_hbm, dst_hbm, alpha_hbm,
           sidx, didx, qbuf, kbuf, abuf, sem):
        wid = lax.axis_index("s") * _NC + lax.axis_index("c")
        base = wid * _EW

        def chunk(ci, _):
            off = base + ci * _C
            pltpu.sync_copy(src_hbm.at[pl.ds(off, _C)], sidx)
            pltpu.sync_copy(dst_hbm.at[pl.ds(off, _C)], didx)
            pltpu.async_copy(q_hbm.at[didx], qbuf, sem).wait()
            pltpu.async_copy(k_hbm.at[sidx], kbuf, sem).wait()

            lanes = lax.broadcasted_iota(jnp.int32, (16,), 0)

            def egroup(eb, _):
                svec = jnp.zeros((16,), jnp.float32)
                for rr in range(16):
                    r = eb * 16 + rr
                    acc = qbuf[r, pl.ds(0, 16)] * kbuf[r, pl.ds(0, 16)]
                    for j in range(1, d // 16):
                        acc = acc + (qbuf[r, pl.ds(16 * j, 16)]
                                     * kbuf[r, pl.ds(16 * j, 16)])
                    tot = jnp.sum(acc, axis=0)
                    svec = jnp.where(lanes == rr, jnp.full((16,), tot), svec)
                abuf[pl.ds(eb * 16, 16)] = svec
                return 0

            lax.fori_loop(0, _C // 16, egroup, 0)
            pltpu.sync_copy(abuf, alpha_hbm.at[pl.ds(off, _C)])
            return 0

        lax.fori_loop(0, _NCHUNK, chunk, 0)

    return _k


def _edge_logits(q, k, src, dst):
    return _make_edge_logits(q.shape[1])(q, k, src, dst)


_NP = 10240
_RPT = _NP // _NS


@functools.lru_cache(maxsize=None)
def _make_edge_scatter(ng, g, isd, with_s):
    cw = 128
    outs = [jax.ShapeDtypeStruct((_NC, _NP, cw), jnp.float32)]
    scratch = [
        pltpu.VMEM((_C,), jnp.int32),
        pltpu.VMEM((_C,), jnp.int32),
        pltpu.VMEM((_C,), jnp.float32),
        pltpu.VMEM((_C, cw), jnp.float32),
        pltpu.VMEM_SHARED((_NP, cw), jnp.float32),
        pltpu.SemaphoreType.DMA,
    ]
    if with_s:
        outs.append(jax.ShapeDtypeStruct((_NC, _NP), jnp.float32))
        scratch.append(pltpu.VMEM_SHARED((_NP,), jnp.float32))

    @functools.partial(
        pl.kernel,
        mesh=_sc_mesh(),
        compiler_params=pltpu.CompilerParams(needs_layout_passes=False),
        out_type=outs,
        scratch_types=scratch,
    )
    def _k(vflat_hbm, src_hbm, dst_hbm, alpha_hbm, m16_hbm, z2_hbm, z1_hbm,
           *refs):
        if with_s:
            agg_hbm, s_hbm, didx, vidx, wbuf, vbuf, acc, sem, acc_s = refs
        else:
            agg_hbm, didx, vidx, wbuf, vbuf, acc, sem = refs
        cid = lax.axis_index("c")
        sid = lax.axis_index("s")
        wid = sid * _NC + cid
        base = wid * _EW
        row0 = sid * _RPT

        pltpu.sync_copy(z2_hbm.at[pl.ds(row0, _RPT), :],
                        acc.at[pl.ds(row0, _RPT), :])
        if with_s:
            pltpu.sync_copy(z1_hbm.at[pl.ds(row0, _RPT)],
                            acc_s.at[pl.ds(row0, _RPT)])
        pltpu.sync_copy(m16_hbm, wbuf.at[pl.ds(0, 16)])
        mvec = wbuf[pl.ds(0, 16)]
        plsc.subcore_barrier()

        def chunk(ci, _):
            off = base + ci * _C
            pltpu.sync_copy(dst_hbm.at[pl.ds(off, _C)], didx)
            pltpu.sync_copy(src_hbm.at[pl.ds(off, _C)], vidx)
            pltpu.sync_copy(alpha_hbm.at[pl.ds(off, _C)], wbuf)
            for b in range(_C // 16):
                if ng > 1:
                    sv = vidx[pl.ds(16 * b, 16)]
                    vidx[pl.ds(16 * b, 16)] = sv * ng + g
                av = wbuf[pl.ds(16 * b, 16)]
                wbuf[pl.ds(16 * b, 16)] = jnp.exp(av * isd - mvec)
            idx_ref = vidx
            pltpu.async_copy(vflat_hbm.at[idx_ref], vbuf, sem).wait()
            for b in range(_C // 16):
                wv = wbuf[pl.ds(16 * b, 16)]
                for rr in range(16):
                    r = 16 * b + rr
                    wr = jnp.full((16,), wv[rr])
                    for j in range(cw // 16):
                        sl = pl.ds(16 * j, 16)
                        vbuf[r, sl] = vbuf[r, sl] * wr
            pltpu.sync_copy(vbuf, acc.at[didx], add=True)
            if with_s:
                pltpu.sync_copy(wbuf, acc_s.at[didx], add=True)
            return 0

        lax.fori_loop(0, _NCHUNK, chunk, 0)
        plsc.subcore_barrier()
        pltpu.sync_copy(acc.at[pl.ds(row0, _RPT), :],
                        agg_hbm.at[cid, pl.ds(row0, _RPT), :])
        if with_s:
            pltpu.sync_copy(acc_s.at[pl.ds(row0, _RPT)],
                            s_hbm.at[cid, pl.ds(row0, _RPT)])

    return _k


def _edge_aggregate(v, src, dst, alpha, d):
    isd = float(1.0 / np.sqrt(np.float64(d)))
    m16 = jnp.full((16,), jnp.max(alpha) * jnp.float32(isd), jnp.float32)
    z2 = jnp.zeros((_NP, 128), jnp.float32)
    z1 = jnp.zeros((_NP,), jnp.float32)
    if d < 128:
        vflat = jnp.pad(v, ((0, 0), (0, 128 - d)))
        ng = 1
    else:
        ng = d // 128
        vflat = v.reshape(_N * ng, 128)
    aggs = []
    s = None
    for g in range(ng):
        fn = _make_edge_scatter(ng, g, isd, g == 0)
        if g == 0:
            agg, sp = fn(vflat, src, dst, alpha, m16, z2, z1)
            s = (sp[0] + sp[1])[:_N]
        else:
            (agg,) = fn(vflat, src, dst, alpha, m16, z2, z1)
        aggs.append((agg[0] + agg[1])[:_N])
    out = jnp.concatenate(aggs, axis=1) if len(aggs) > 1 else aggs[0]
    return out[:, :d], s


def _linear_body(x_ref, w_ref, b_ref, o_ref):
    o_ref[...] = (
        jnp.dot(x_ref[...], w_ref[...], preferred_element_type=jnp.float32)
        + b_ref[...]
    )


def _linear(x, Wc, bc):
    n, fin = x.shape
    fout = Wc.shape[1]
    grid = (n // _BN,)
    return pl.pallas_call(
        _linear_body,
        grid=grid,
        in_specs=[
            pl.BlockSpec((_BN, fin), lambda i: (i, 0)),
            pl.BlockSpec((fin, fout), lambda i: (0, 0)),
            pl.BlockSpec((1, fout), lambda i: (0, 0)),
        ],
        out_specs=pl.BlockSpec((_BN, fout), lambda i: (i, 0)),
        out_shape=jax.ShapeDtypeStruct((n, fout), jnp.float32),
    )(x, Wc, bc)


def _pool_body(h_ref, b1h_ref, gw_ref, gb_ref, fw_ref, fb_ref, o_ref):
    h = h_ref[...]
    gate = jnp.dot(h, gw_ref[...], preferred_element_type=jnp.float32)
    gate = gate + gb_ref[0, 0]
    seg = b1h_ref[...]
    gid = jax.lax.broadcasted_iota(jnp.int32, (_N, _G), 1)
    mask = gid == seg
    neg = jnp.float32(-1e30)
    m = jnp.max(jnp.where(mask, gate, neg), axis=0)
    m = jnp.where(m > jnp.float32(-1e29), m, 0.0)
    e = jnp.exp(gate - m[None, :]) * mask.astype(jnp.float32)
    s = jnp.sum(e, axis=0)
    w = e / (s[None, :] + 1e-16)
    pooled = jax.lax.dot_general(
        w, h, (((0,), (0,)), ((), ())),
        preferred_element_type=jnp.float32)
    o_ref[...] = (
        jnp.dot(pooled, fw_ref[...], preferred_element_type=jnp.float32)
        + fb_ref[...]
    )


def _pool_head(h, batch, gate_W, gate_b, fc_W, fc_b):
    b1h = batch.reshape(_N, 1)
    return pl.pallas_call(
        _pool_body,
        out_shape=jax.ShapeDtypeStruct((_G, 2), jnp.float32),
    )(h, b1h, gate_W.T, gate_b.reshape(1, 1), fc_W.T, fc_b.reshape(1, 2))


def _edge_phase(q, k, v, src, dst):
    d = q.shape[1]
    if d < 128:
        pad = ((0, 0), (0, 128 - d))
        qp, kp = jnp.pad(q, pad), jnp.pad(k, pad)
    else:
        qp, kp = q, k
    alpha = _edge_logits(qp, kp, src, dst)
    return _edge_aggregate(v, src, dst, alpha, d)


def kernel(x, edge_index, batch, params):
    src, dst = edge_index[0], edge_index[1]
    h = x
    for i in range(1, 5):
        p = params
        Wc = jnp.concatenate(
            [p["conv%d_W%s" % (i, n)].T for n in ("q", "k", "v", "s")], axis=1
        )
        bc = jnp.concatenate(
            [p["conv%d_b%s" % (i, n)] for n in ("q", "k", "v", "s")]
        ).reshape(1, -1)
        qkvs = _linear(h, Wc, bc)
        d = Wc.shape[1] // 4
        q, k_, v, skip = (qkvs[:, j * d:(j + 1) * d] for j in range(4))
        agg, s = _edge_phase(q, k_, v, src, dst)
        h = jax.nn.elu(agg / (s[:, None] + 1e-16) + skip)
    return _pool_head(
        h, batch, params["gate_W"], params["gate_b"],
        params["fc_W"], params["fc_b"],
    )


# speedup vs baseline: 4.8661x; 1.2678x over previous
import functools

import jax
import jax.numpy as jnp
import numpy as np
from jax import lax
from jax.experimental import pallas as pl
from jax.experimental.pallas import tpu as pltpu
from jax.experimental.pallas import tpu_sc as plsc

_N = 10000
_G = 8
_E = 320000
_BN = 1000

_NC = 2
_NS = 16
_NW = _NC * _NS
_EW = _E // _NW
_C = 80
_NCHUNK = _EW // _C


def _sc_mesh():
    return plsc.VectorSubcoreMesh(core_axis_name="c", subcore_axis_name="s")


_C1 = 16
_NCH1 = _EW // _C1


@functools.lru_cache(maxsize=None)
def _make_edge_logits(d):
    @functools.partial(
        pl.kernel,
        mesh=_sc_mesh(),
        compiler_params=pltpu.CompilerParams(needs_layout_passes=False),
        out_type=jax.ShapeDtypeStruct((_E,), jnp.float32),
        scratch_types=[
            pltpu.VMEM((_EW,), jnp.int32),
            pltpu.VMEM((_EW,), jnp.int32),
            pltpu.VMEM((_C1, d), jnp.float32),
            pltpu.VMEM((_C1, d), jnp.float32),
            pltpu.VMEM((_C1, d), jnp.float32),
            pltpu.VMEM((_C1, d), jnp.float32),
            pltpu.VMEM((16,), jnp.float32),
            pltpu.VMEM((16,), jnp.float32),
            pltpu.SemaphoreType.DMA,
            pltpu.SemaphoreType.DMA,
            pltpu.SemaphoreType.DMA,
            pltpu.SemaphoreType.DMA,
            pltpu.SemaphoreType.DMA,
            pltpu.SemaphoreType.DMA,
        ],
    )
    def _k(q_hbm, k_hbm, src_hbm, dst_hbm, alpha_hbm,
           sidx, didx, qbA, kbA, qbB, kbB, abA, abB,
           sqA, skA, sqB, skB, saA, saB):
        wid = lax.axis_index("s") * _NC + lax.axis_index("c")
        base = wid * _EW
        lanes = lax.broadcasted_iota(jnp.int32, (16,), 0)

        pltpu.sync_copy(src_hbm.at[pl.ds(base, _EW)], sidx)
        pltpu.sync_copy(dst_hbm.at[pl.ds(base, _EW)], didx)

        def start(ci, qb, kb, sq, sk):
            sl = pl.ds(ci * _C1, _C1)
            pltpu.async_copy(q_hbm.at[didx.at[sl]], qb, sq)
            pltpu.async_copy(k_hbm.at[sidx.at[sl]], kb, sk)

        def wait(ci, qb, kb, sq, sk):
            sl = pl.ds(ci * _C1, _C1)
            pltpu.make_async_copy(q_hbm.at[didx.at[sl]], qb, sq).wait()
            pltpu.make_async_copy(k_hbm.at[sidx.at[sl]], kb, sk).wait()

        def out_slot(ci):
            return alpha_hbm.at[pl.ds(base + ci * _C1, 16)]

        def compute(ci, qb, kb, ab, sa):
            svec = jnp.zeros((16,), jnp.float32)
            for rr in range(16):
                acc = qb[rr, pl.ds(0, 16)] * kb[rr, pl.ds(0, 16)]
                for j in range(1, d // 16):
                    acc = acc + (qb[rr, pl.ds(16 * j, 16)]
                                 * kb[rr, pl.ds(16 * j, 16)])
                tot = jnp.sum(acc, axis=0)
                svec = jnp.where(lanes == rr, jnp.full((16,), tot), svec)
            ab[...] = svec
            pltpu.async_copy(ab, out_slot(ci), sa)

        start(0, qbA, kbA, sqA, skA)

        def body(i, _):
            c0 = 2 * i
            c1 = c0 + 1
            start(c1, qbB, kbB, sqB, skB)
            wait(c0, qbA, kbA, sqA, skA)

            @pl.when(i > 0)
            def _():
                pltpu.make_async_copy(abA, out_slot(c0 - 2), saA).wait()
                pltpu.make_async_copy(abB, out_slot(c0 - 1), saB).wait()

            compute(c0, qbA, kbA, abA, saA)
            start(c0 + 2, qbA, kbA, sqA, skA)
            wait(c1, qbB, kbB, sqB, skB)
            compute(c1, qbB, kbB, abB, saB)
            return 0

        lax.fori_loop(0, _NCH1 // 2, body, 0)
        last = _NCH1 - 1
        wait(last, qbA, kbA, sqA, skA)
        pltpu.make_async_copy(abA, out_slot(last - 2), saA).wait()
        pltpu.make_async_copy(abB, out_slot(last - 1), saB).wait()
        compute(last, qbA, kbA, abA, saA)
        pltpu.make_async_copy(abA, out_slot(last), saA).wait()

    return _k


def _edge_logits(q, k, src, dst):
    return _make_edge_logits(q.shape[1])(q, k, src, dst)


_NP = 10240
_RPT = _NP // _NS


@functools.lru_cache(maxsize=None)
def _make_edge_scatter(ng, g, isd, with_s):
    cw = 128
    outs = [jax.ShapeDtypeStruct((_NC, _NP, cw), jnp.float32)]
    scratch = [
        pltpu.VMEM((_C,), jnp.int32),
        pltpu.VMEM((_C,), jnp.int32),
        pltpu.VMEM((_C,), jnp.float32),
        pltpu.VMEM((_C, cw), jnp.float32),
        pltpu.VMEM_SHARED((_NP, cw), jnp.float32),
        pltpu.SemaphoreType.DMA,
    ]
    if with_s:
        outs.append(jax.ShapeDtypeStruct((_NC, _NP), jnp.float32))
        scratch.append(pltpu.VMEM_SHARED((_NP,), jnp.float32))

    @functools.partial(
        pl.kernel,
        mesh=_sc_mesh(),
        compiler_params=pltpu.CompilerParams(needs_layout_passes=False),
        out_type=outs,
        scratch_types=scratch,
    )
    def _k(vflat_hbm, src_hbm, dst_hbm, alpha_hbm, m16_hbm, z2_hbm, z1_hbm,
           *refs):
        if with_s:
            agg_hbm, s_hbm, didx, vidx, wbuf, vbuf, acc, sem, acc_s = refs
        else:
            agg_hbm, didx, vidx, wbuf, vbuf, acc, sem = refs
        cid = lax.axis_index("c")
        sid = lax.axis_index("s")
        wid = sid * _NC + cid
        base = wid * _EW
        row0 = sid * _RPT

        pltpu.sync_copy(z2_hbm.at[pl.ds(row0, _RPT), :],
                        acc.at[pl.ds(row0, _RPT), :])
        if with_s:
            pltpu.sync_copy(z1_hbm.at[pl.ds(row0, _RPT)],
                            acc_s.at[pl.ds(row0, _RPT)])
        pltpu.sync_copy(m16_hbm, wbuf.at[pl.ds(0, 16)])
        mvec = wbuf[pl.ds(0, 16)]
        plsc.subcore_barrier()

        def chunk(ci, _):
            off = base + ci * _C
            pltpu.sync_copy(dst_hbm.at[pl.ds(off, _C)], didx)
            pltpu.sync_copy(src_hbm.at[pl.ds(off, _C)], vidx)
            pltpu.sync_copy(alpha_hbm.at[pl.ds(off, _C)], wbuf)
            for b in range(_C // 16):
                if ng > 1:
                    sv = vidx[pl.ds(16 * b, 16)]
                    vidx[pl.ds(16 * b, 16)] = sv * ng + g
                av = wbuf[pl.ds(16 * b, 16)]
                wbuf[pl.ds(16 * b, 16)] = jnp.exp(av * isd - mvec)
            idx_ref = vidx
            pltpu.async_copy(vflat_hbm.at[idx_ref], vbuf, sem).wait()
            for b in range(_C // 16):
                wv = wbuf[pl.ds(16 * b, 16)]
                for rr in range(16):
                    r = 16 * b + rr
                    wr = jnp.full((16,), wv[rr])
                    for j in range(cw // 16):
                        sl = pl.ds(16 * j, 16)
                        vbuf[r, sl] = vbuf[r, sl] * wr
            pltpu.sync_copy(vbuf, acc.at[didx], add=True)
            if with_s:
                pltpu.sync_copy(wbuf, acc_s.at[didx], add=True)
            return 0

        lax.fori_loop(0, _NCHUNK, chunk, 0)
        plsc.subcore_barrier()
        pltpu.sync_copy(acc.at[pl.ds(row0, _RPT), :],
                        agg_hbm.at[cid, pl.ds(row0, _RPT), :])
        if with_s:
            pltpu.sync_copy(acc_s.at[pl.ds(row0, _RPT)],
                            s_hbm.at[cid, pl.ds(row0, _RPT)])

    return _k


def _edge_aggregate(v, src, dst, alpha, d):
    isd = float(1.0 / np.sqrt(np.float64(d)))
    m16 = jnp.full((16,), jnp.max(alpha) * jnp.float32(isd), jnp.float32)
    z2 = jnp.zeros((_NP, 128), jnp.float32)
    z1 = jnp.zeros((_NP,), jnp.float32)
    if d < 128:
        vflat = jnp.pad(v, ((0, 0), (0, 128 - d)))
        ng = 1
    else:
        ng = d // 128
        vflat = v.reshape(_N * ng, 128)
    aggs = []
    s = None
    for g in range(ng):
        fn = _make_edge_scatter(ng, g, isd, g == 0)
        if g == 0:
            agg, sp = fn(vflat, src, dst, alpha, m16, z2, z1)
            s = (sp[0] + sp[1])[:_N]
        else:
            (agg,) = fn(vflat, src, dst, alpha, m16, z2, z1)
        aggs.append((agg[0] + agg[1])[:_N])
    out = jnp.concatenate(aggs, axis=1) if len(aggs) > 1 else aggs[0]
    return out[:, :d], s


def _linear_body(x_ref, w_ref, b_ref, o_ref):
    o_ref[...] = (
        jnp.dot(x_ref[...], w_ref[...], preferred_element_type=jnp.float32)
        + b_ref[...]
    )


def _linear(x, Wc, bc):
    n, fin = x.shape
    fout = Wc.shape[1]
    grid = (n // _BN,)
    return pl.pallas_call(
        _linear_body,
        grid=grid,
        in_specs=[
            pl.BlockSpec((_BN, fin), lambda i: (i, 0)),
            pl.BlockSpec((fin, fout), lambda i: (0, 0)),
            pl.BlockSpec((1, fout), lambda i: (0, 0)),
        ],
        out_specs=pl.BlockSpec((_BN, fout), lambda i: (i, 0)),
        out_shape=jax.ShapeDtypeStruct((n, fout), jnp.float32),
    )(x, Wc, bc)


def _pool_body(h_ref, b1h_ref, gw_ref, gb_ref, fw_ref, fb_ref, o_ref):
    h = h_ref[...]
    gate = jnp.dot(h, gw_ref[...], preferred_element_type=jnp.float32)
    gate = gate + gb_ref[0, 0]
    seg = b1h_ref[...]
    gid = jax.lax.broadcasted_iota(jnp.int32, (_N, _G), 1)
    mask = gid == seg
    neg = jnp.float32(-1e30)
    m = jnp.max(jnp.where(mask, gate, neg), axis=0)
    m = jnp.where(m > jnp.float32(-1e29), m, 0.0)
    e = jnp.exp(gate - m[None, :]) * mask.astype(jnp.float32)
    s = jnp.sum(e, axis=0)
    w = e / (s[None, :] + 1e-16)
    pooled = jax.lax.dot_general(
        w, h, (((0,), (0,)), ((), ())),
        preferred_element_type=jnp.float32)
    o_ref[...] = (
        jnp.dot(pooled, fw_ref[...], preferred_element_type=jnp.float32)
        + fb_ref[...]
    )


def _pool_head(h, batch, gate_W, gate_b, fc_W, fc_b):
    b1h = batch.reshape(_N, 1)
    return pl.pallas_call(
        _pool_body,
        out_shape=jax.ShapeDtypeStruct((_G, 2), jnp.float32),
    )(h, b1h, gate_W.T, gate_b.reshape(1, 1), fc_W.T, fc_b.reshape(1, 2))


def _edge_phase(q, k, v, src, dst):
    d = q.shape[1]
    if d < 128:
        pad = ((0, 0), (0, 128 - d))
        qp, kp = jnp.pad(q, pad), jnp.pad(k, pad)
    else:
        qp, kp = q, k
    alpha = _edge_logits(qp, kp, src, dst)
    return _edge_aggregate(v, src, dst, alpha, d)


def kernel(x, edge_index, batch, params):
    src, dst = edge_index[0], edge_index[1]
    h = x
    for i in range(1, 5):
        p = params
        Wc = jnp.concatenate(
            [p["conv%d_W%s" % (i, n)].T for n in ("q", "k", "v", "s")], axis=1
        )
        bc = jnp.concatenate(
            [p["conv%d_b%s" % (i, n)] for n in ("q", "k", "v", "s")]
        ).reshape(1, -1)
        qkvs = _linear(h, Wc, bc)
        d = Wc.shape[1] // 4
        q, k_, v, skip = (qkvs[:, j * d:(j + 1) * d] for j in range(4))
        agg, s = _edge_phase(q, k_, v, src, dst)
        h = jax.nn.elu(agg / (s[:, None] + 1e-16) + skip)
    return _pool_head(
        h, batch, params["gate_W"], params["gate_b"],
        params["fc_W"], params["fc_b"],
    )


# speedup vs baseline: 6.1672x; 1.2674x over previous
import functools

import jax
import jax.numpy as jnp
import numpy as np
from jax import lax
from jax.experimental import pallas as pl
from jax.experimental.pallas import tpu as pltpu
from jax.experimental.pallas import tpu_sc as plsc

_N = 10000
_G = 8
_E = 320000
_BN = 1000

_NC = 2
_NS = 16
_NW = _NC * _NS
_EW = _E // _NW
_C = 80
_NCHUNK = _EW // _C


def _sc_mesh():
    return plsc.VectorSubcoreMesh(core_axis_name="c", subcore_axis_name="s")


_C1 = 16
_NCH1 = _EW // _C1


@functools.lru_cache(maxsize=None)
def _make_edge_logits(d):
    @functools.partial(
        pl.kernel,
        mesh=_sc_mesh(),
        compiler_params=pltpu.CompilerParams(needs_layout_passes=False),
        out_type=jax.ShapeDtypeStruct((_E,), jnp.float32),
        scratch_types=[
            pltpu.VMEM((_EW,), jnp.int32),
            pltpu.VMEM((_EW,), jnp.int32),
            pltpu.VMEM((_C1, d), jnp.float32),
            pltpu.VMEM((_C1, d), jnp.float32),
            pltpu.VMEM((_C1, d), jnp.float32),
            pltpu.VMEM((_C1, d), jnp.float32),
            pltpu.VMEM((16,), jnp.float32),
            pltpu.VMEM((16,), jnp.float32),
            pltpu.SemaphoreType.DMA,
            pltpu.SemaphoreType.DMA,
            pltpu.SemaphoreType.DMA,
            pltpu.SemaphoreType.DMA,
            pltpu.SemaphoreType.DMA,
            pltpu.SemaphoreType.DMA,
        ],
    )
    def _k(q_hbm, k_hbm, src_hbm, dst_hbm, alpha_hbm,
           sidx, didx, qbA, kbA, qbB, kbB, abA, abB,
           sqA, skA, sqB, skB, saA, saB):
        wid = lax.axis_index("s") * _NC + lax.axis_index("c")
        base = wid * _EW
        lanes = lax.broadcasted_iota(jnp.int32, (16,), 0)

        pltpu.sync_copy(src_hbm.at[pl.ds(base, _EW)], sidx)
        pltpu.sync_copy(dst_hbm.at[pl.ds(base, _EW)], didx)

        def start(ci, qb, kb, sq, sk):
            sl = pl.ds(ci * _C1, _C1)
            pltpu.async_copy(q_hbm.at[didx.at[sl]], qb, sq)
            pltpu.async_copy(k_hbm.at[sidx.at[sl]], kb, sk)

        def wait(ci, qb, kb, sq, sk):
            sl = pl.ds(ci * _C1, _C1)
            pltpu.make_async_copy(q_hbm.at[didx.at[sl]], qb, sq).wait()
            pltpu.make_async_copy(k_hbm.at[sidx.at[sl]], kb, sk).wait()

        def out_slot(ci):
            return alpha_hbm.at[pl.ds(base + ci * _C1, 16)]

        def compute(ci, qb, kb, ab, sa):
            svec = jnp.zeros((16,), jnp.float32)
            for rr in range(16):
                acc = qb[rr, pl.ds(0, 16)] * kb[rr, pl.ds(0, 16)]
                for j in range(1, d // 16):
                    acc = acc + (qb[rr, pl.ds(16 * j, 16)]
                                 * kb[rr, pl.ds(16 * j, 16)])
                tot = jnp.sum(acc, axis=0)
                svec = jnp.where(lanes == rr, jnp.full((16,), tot), svec)
            ab[...] = svec
            pltpu.async_copy(ab, out_slot(ci), sa)

        start(0, qbA, kbA, sqA, skA)

        def body(i, _):
            c0 = 2 * i
            c1 = c0 + 1
            start(c1, qbB, kbB, sqB, skB)
            wait(c0, qbA, kbA, sqA, skA)

            @pl.when(i > 0)
            def _():
                pltpu.make_async_copy(abA, out_slot(c0 - 2), saA).wait()
                pltpu.make_async_copy(abB, out_slot(c0 - 1), saB).wait()

            compute(c0, qbA, kbA, abA, saA)
            start(c0 + 2, qbA, kbA, sqA, skA)
            wait(c1, qbB, kbB, sqB, skB)
            compute(c1, qbB, kbB, abB, saB)
            return 0

        lax.fori_loop(0, _NCH1 // 2, body, 0)
        last = _NCH1 - 1
        wait(last, qbA, kbA, sqA, skA)
        pltpu.make_async_copy(abA, out_slot(last - 2), saA).wait()
        pltpu.make_async_copy(abB, out_slot(last - 1), saB).wait()
        compute(last, qbA, kbA, abA, saA)
        pltpu.make_async_copy(abA, out_slot(last), saA).wait()

    return _k


def _edge_logits(q, k, src, dst):
    return _make_edge_logits(q.shape[1])(q, k, src, dst)


_NP = 10240
_RPT = _NP // _NS


_C2 = 40
_NCH2 = _EW // _C2
_OFFS = (0, 16, 24)


@functools.lru_cache(maxsize=None)
def _make_edge_scatter(ng, g, isd, with_s):
    cw = 128
    outs = [jax.ShapeDtypeStruct((_NC, _NP, cw), jnp.float32)]
    scratch = [
        pltpu.VMEM((3, _C2), jnp.int32),
        pltpu.VMEM((3, _C2), jnp.int32),
        pltpu.VMEM((_C2,), jnp.int32),
        pltpu.VMEM((_C2,), jnp.int32),
        pltpu.VMEM((_C2,), jnp.float32),
        pltpu.VMEM((_C2,), jnp.float32),
        pltpu.VMEM((_C2, cw), jnp.float32),
        pltpu.VMEM((_C2, cw), jnp.float32),
        pltpu.VMEM((1, _C2), jnp.int32),
        pltpu.VMEM((1, _C2), jnp.int32),
        pltpu.VMEM((16,), jnp.float32),
        pltpu.VMEM_SHARED((_NP, cw), jnp.float32),
        pltpu.SemaphoreType.DMA,
        pltpu.SemaphoreType.DMA,
        pltpu.SemaphoreType.DMA,
        pltpu.SemaphoreType.DMA,
        pltpu.SemaphoreType.DMA,
        pltpu.SemaphoreType.DMA,
    ]
    if with_s:
        outs.append(jax.ShapeDtypeStruct((_NC, _NP), jnp.float32))
        scratch.append(pltpu.VMEM_SHARED((_NP,), jnp.float32))

    @functools.partial(
        pl.kernel,
        mesh=_sc_mesh(),
        compiler_params=pltpu.CompilerParams(needs_layout_passes=False),
        out_type=outs,
        scratch_types=scratch,
    )
    def _k(vflat_hbm, pk_hbm, m16_hbm, z2_hbm, z1_hbm, *refs):
        if with_s:
            (agg_hbm, s_hbm, pA, pB, viA, viB, wbA, wbB, vbA, vbB,
             diA, diB, mbuf, acc, spA, spB, sgA, sgB, ssA, ssB,
             acc_s) = refs
        else:
            (agg_hbm, pA, pB, viA, viB, wbA, wbB, vbA, vbB,
             diA, diB, mbuf, acc, spA, spB, sgA, sgB, ssA, ssB) = refs
        cid = lax.axis_index("c")
        sid = lax.axis_index("s")
        wid = sid * _NC + cid
        row0 = sid * _RPT
        lanes = lax.broadcasted_iota(jnp.int32, (16,), 0)

        pltpu.sync_copy(z2_hbm.at[pl.ds(row0, _RPT), :],
                        acc.at[pl.ds(row0, _RPT), :])
        if with_s:
            pltpu.sync_copy(z1_hbm.at[pl.ds(row0, _RPT)],
                            acc_s.at[pl.ds(row0, _RPT)])
        pltpu.sync_copy(m16_hbm, mbuf)
        mvec = mbuf[...]
        plsc.subcore_barrier()

        def start_packed(ci, p, sp):
            pltpu.async_copy(pk_hbm.at[wid, ci], p, sp)

        def wait_packed(ci, p, sp):
            pltpu.make_async_copy(pk_hbm.at[wid, ci], p, sp).wait()

        def prep_gather(ci, p, vi, sg, vb):
            for o in _OFFS:
                sv = p[0, pl.ds(o, 16)]
                vi[pl.ds(o, 16)] = sv * ng + g if ng > 1 else sv
            pltpu.async_copy(vflat_hbm.at[vi], vb, sg)

        def wait_gather(ci, vi, sg, vb):
            pltpu.make_async_copy(vflat_hbm.at[vi], vb, sg).wait()

        def scale_and_scatter(ci, p, wb, vb, di, ss):
            for o in _OFFS:
                di[0, pl.ds(o, 16)] = p[1, pl.ds(o, 16)]
                ab = plsc.bitcast(p[2, pl.ds(o, 16)], jnp.float32)
                wv = jnp.exp(ab * isd - mvec)
                wb[pl.ds(o, 16)] = wv
                lo = 8 if o == 24 else 0
                for rr in range(lo, 16):
                    r = o + rr
                    wr = jnp.full((16,), wv[rr])
                    for j in range(cw // 16):
                        sl = pl.ds(16 * j, 16)
                        vb[r, sl] = vb[r, sl] * wr
            pltpu.async_copy(vb, acc.at[di.at[0]], ss, add=True)
            if with_s:
                pltpu.async_copy(wb, acc_s.at[di.at[0]], ss, add=True)

        def wait_scatter(ci, wb, vb, di, ss):
            pltpu.make_async_copy(vb, acc.at[di.at[0]], ss).wait()
            if with_s:
                pltpu.make_async_copy(wb, acc_s.at[di.at[0]], ss).wait()

        start_packed(0, pA, spA)
        start_packed(1, pB, spB)
        wait_packed(0, pA, spA)
        prep_gather(0, pA, viA, sgA, vbA)

        def body(i, _):
            c0 = 2 * i
            c1 = c0 + 1
            wait_packed(c1, pB, spB)
            prep_gather(c1, pB, viB, sgB, vbB)
            wait_gather(c0, viA, sgA, vbA)

            @pl.when(i > 0)
            def _():
                wait_scatter(c0 - 2, wbA, vbA, diA, ssA)

            scale_and_scatter(c0, pA, wbA, vbA, diA, ssA)

            @pl.when(i < _NCH2 // 2 - 1)
            def _():
                start_packed(c0 + 2, pA, spA)

            wait_gather(c1, viB, sgB, vbB)

            @pl.when(i > 0)
            def _():
                wait_scatter(c1 - 2, wbB, vbB, diB, ssB)

            scale_and_scatter(c1, pB, wbB, vbB, diB, ssB)

            @pl.when(i < _NCH2 // 2 - 1)
            def _():
                start_packed(c1 + 2, pB, spB)
                wait_packed(c0 + 2, pA, spA)
                prep_gather(c0 + 2, pA, viA, sgA, vbA)

            return 0

        lax.fori_loop(0, _NCH2 // 2, body, 0)
        wait_scatter(_NCH2 - 2, wbA, vbA, diA, ssA)
        wait_scatter(_NCH2 - 1, wbB, vbB, diB, ssB)
        plsc.subcore_barrier()
        pltpu.sync_copy(acc.at[pl.ds(row0, _RPT), :],
                        agg_hbm.at[cid, pl.ds(row0, _RPT), :])
        if with_s:
            pltpu.sync_copy(acc_s.at[pl.ds(row0, _RPT)],
                            s_hbm.at[cid, pl.ds(row0, _RPT)])

    return _k


def _edge_aggregate(v, src, dst, alpha, d):
    isd = float(1.0 / np.sqrt(np.float64(d)))
    m16 = jnp.full((16,), jnp.max(alpha) * jnp.float32(isd), jnp.float32)
    z2 = jnp.zeros((_NP, 128), jnp.float32)
    z1 = jnp.zeros((_NP,), jnp.float32)
    pk = jnp.stack(
        [src.reshape(_NW, _NCH2, _C2),
         dst.reshape(_NW, _NCH2, _C2),
         jax.lax.bitcast_convert_type(alpha, jnp.int32).reshape(
             _NW, _NCH2, _C2)],
        axis=2)
    if d < 128:
        vflat = jnp.pad(v, ((0, 0), (0, 128 - d)))
        ng = 1
    else:
        ng = d // 128
        vflat = v.reshape(_N * ng, 128)
    aggs = []
    s = None
    for g in range(ng):
        fn = _make_edge_scatter(ng, g, isd, g == 0)
        if g == 0:
            agg, sp = fn(vflat, pk, m16, z2, z1)
            s = (sp[0] + sp[1])[:_N]
        else:
            (agg,) = fn(vflat, pk, m16, z2, z1)
        aggs.append((agg[0] + agg[1])[:_N])
    out = jnp.concatenate(aggs, axis=1) if len(aggs) > 1 else aggs[0]
    return out[:, :d], s


def _linear_body(x_ref, w_ref, b_ref, o_ref):
    o_ref[...] = (
        jnp.dot(x_ref[...], w_ref[...], preferred_element_type=jnp.float32)
        + b_ref[...]
    )


def _linear(x, Wc, bc):
    n, fin = x.shape
    fout = Wc.shape[1]
    grid = (n // _BN,)
    return pl.pallas_call(
        _linear_body,
        grid=grid,
        in_specs=[
            pl.BlockSpec((_BN, fin), lambda i: (i, 0)),
            pl.BlockSpec((fin, fout), lambda i: (0, 0)),
            pl.BlockSpec((1, fout), lambda i: (0, 0)),
        ],
        out_specs=pl.BlockSpec((_BN, fout), lambda i: (i, 0)),
        out_shape=jax.ShapeDtypeStruct((n, fout), jnp.float32),
    )(x, Wc, bc)


def _pool_body(h_ref, b1h_ref, gw_ref, gb_ref, fw_ref, fb_ref, o_ref):
    h = h_ref[...]
    gate = jnp.dot(h, gw_ref[...], preferred_element_type=jnp.float32)
    gate = gate + gb_ref[0, 0]
    seg = b1h_ref[...]
    gid = jax.lax.broadcasted_iota(jnp.int32, (_N, _G), 1)
    mask = gid == seg
    neg = jnp.float32(-1e30)
    m = jnp.max(jnp.where(mask, gate, neg), axis=0)
    m = jnp.where(m > jnp.float32(-1e29), m, 0.0)
    e = jnp.exp(gate - m[None, :]) * mask.astype(jnp.float32)
    s = jnp.sum(e, axis=0)
    w = e / (s[None, :] + 1e-16)
    pooled = jax.lax.dot_general(
        w, h, (((0,), (0,)), ((), ())),
        preferred_element_type=jnp.float32)
    o_ref[...] = (
        jnp.dot(pooled, fw_ref[...], preferred_element_type=jnp.float32)
        + fb_ref[...]
    )


def _pool_head(h, batch, gate_W, gate_b, fc_W, fc_b):
    b1h = batch.reshape(_N, 1)
    return pl.pallas_call(
        _pool_body,
        out_shape=jax.ShapeDtypeStruct((_G, 2), jnp.float32),
    )(h, b1h, gate_W.T, gate_b.reshape(1, 1), fc_W.T, fc_b.reshape(1, 2))


def _edge_phase(q, k, v, src, dst):
    d = q.shape[1]
    if d < 128:
        pad = ((0, 0), (0, 128 - d))
        qp, kp = jnp.pad(q, pad), jnp.pad(k, pad)
    else:
        qp, kp = q, k
    alpha = _edge_logits(qp, kp, src, dst)
    return _edge_aggregate(v, src, dst, alpha, d)


def kernel(x, edge_index, batch, params):
    src, dst = edge_index[0], edge_index[1]
    h = x
    for i in range(1, 5):
        p = params
        Wc = jnp.concatenate(
            [p["conv%d_W%s" % (i, n)].T for n in ("q", "k", "v", "s")], axis=1
        )
        bc = jnp.concatenate(
            [p["conv%d_b%s" % (i, n)] for n in ("q", "k", "v", "s")]
        ).reshape(1, -1)
        qkvs = _linear(h, Wc, bc)
        d = Wc.shape[1] // 4
        q, k_, v, skip = (qkvs[:, j * d:(j + 1) * d] for j in range(4))
        agg, s = _edge_phase(q, k_, v, src, dst)
        h = jax.nn.elu(agg / (s[:, None] + 1e-16) + skip)
    return _pool_head(
        h, batch, params["gate_W"], params["gate_b"],
        params["fc_W"], params["fc_b"],
    )
